# baseline (speedup 1.0000x reference)
import jax
import jax.numpy as jnp
from jax import lax
from jax.experimental import pallas as pl
from jax.experimental.pallas import tpu as pltpu
from jax.experimental.pallas import tpu_sc as plsc

N = 10000
E = 320000
D = 128
G = 64
NC = 2
NS = 16
NW = NC * NS
CH = 128
NCHUNK = 81
EPW = CH * NCHUNK
EP = NW * EPW
NPASS = 2
NPAD = 10240
HALF = NPAD // 2
ACCR = HALF + 8
RPC = NPAD // NS
RPS = HALF // NS

_f32 = jnp.float32
_i32 = jnp.int32

_MESH = plsc.VectorSubcoreMesh(core_axis_name="c", subcore_axis_name="s")


def _wid():
    return lax.axis_index("c") * NS + lax.axis_index("s")


def _deg_body(dst_hbm, ew_hbm, out_hbm, dst_v, ew_v, zrow_v, acc_sh):
    cid = lax.axis_index("c")
    sid = lax.axis_index("s")
    wid = _wid()

    def zrow(i, c):
        zrow_v[pl.ds(pl.multiple_of(i * 16, 16), 16)] = jnp.zeros((16,), _f32)
        return c

    lax.fori_loop(0, RPC // 16, zrow, 0)
    pltpu.sync_copy(zrow_v, acc_sh.at[pl.ds(sid * RPC, RPC)])
    plsc.subcore_barrier()

    pltpu.sync_copy(dst_hbm.at[wid], dst_v)
    pltpu.sync_copy(ew_hbm.at[wid], ew_v)

    def chunk(c, carry):
        pltpu.sync_copy(ew_v.at[c], acc_sh.at[dst_v.at[c]], add=True)
        return carry

    lax.fori_loop(0, NCHUNK, chunk, 0)
    plsc.subcore_barrier()
    pltpu.sync_copy(acc_sh.at[pl.ds(sid * RPC, RPC)],
                    out_hbm.at[cid, pl.ds(sid * RPC, RPC)])


_deg_call = pl.kernel(
    _deg_body,
    out_type=jax.ShapeDtypeStruct((NC, NPAD), _f32),
    mesh=_MESH,
    scratch_types=[
        pltpu.VMEM((NCHUNK, CH), _i32),
        pltpu.VMEM((NCHUNK, CH), _f32),
        pltpu.VMEM((RPC,), _f32),
        pltpu.VMEM_SHARED((NPAD,), _f32),
    ],
)


def _norm_body(dinv_hbm, src_hbm, dst_hbm, ew_hbm, norm_hbm,
               src_v, dst_v, ew_v, norm_v, ds_v, dd_v, sem, sem2):
    wid = _wid()
    pltpu.sync_copy(src_hbm.at[wid], src_v)
    pltpu.sync_copy(dst_hbm.at[wid], dst_v)
    pltpu.sync_copy(ew_hbm.at[wid], ew_v)

    def chunk(c, carry):
        a = pltpu.async_copy(dinv_hbm.at[src_v.at[c]], ds_v, sem)
        b = pltpu.async_copy(dinv_hbm.at[dst_v.at[c]], dd_v, sem2)
        a.wait()
        b.wait()
        for j in range(CH // 16):
            sl = pl.ds(j * 16, 16)
            norm_v[c, sl] = ds_v[sl] * ew_v[c, sl] * dd_v[sl]
        return carry

    lax.fori_loop(0, NCHUNK, chunk, 0)
    pltpu.sync_copy(norm_v, norm_hbm.at[wid])


_norm_call = pl.kernel(
    _norm_body,
    out_type=jax.ShapeDtypeStruct((NW, NCHUNK, CH), _f32),
    mesh=_MESH,
    scratch_types=[
        pltpu.VMEM((NCHUNK, CH), _i32),
        pltpu.VMEM((NCHUNK, CH), _i32),
        pltpu.VMEM((NCHUNK, CH), _f32),
        pltpu.VMEM((NCHUNK, CH), _f32),
        pltpu.VMEM((CH,), _f32),
        pltpu.VMEM((CH,), _f32),
        pltpu.SemaphoreType.DMA,
        pltpu.SemaphoreType.DMA,
    ],
)


def _msg_body(g_hbm, src_hbm, dst_hbm, norm_hbm, out_hbm,
              src_v, dst_v, norm_v, rows_v, rows2_v, acc_sh, sem, sem2,
              ssem, ssem2):
    cid = lax.axis_index("c")
    sid = lax.axis_index("s")

    def zrow(r, c):
        for j in range(D // 16):
            rows_v[r, pl.ds(j * 16, 16)] = jnp.zeros((16,), _f32)
        return c

    lax.fori_loop(0, CH, zrow, 0)
    for k in range(2):
        pltpu.sync_copy(rows_v, acc_sh.at[pl.ds(sid * RPS + k * CH, CH)])
    pltpu.sync_copy(rows_v.at[pl.ds(0, RPS - 2 * CH)],
                    acc_sh.at[pl.ds(sid * RPS + 2 * CH, RPS - 2 * CH)])

    @pl.when(sid == 0)
    def _():
        pltpu.sync_copy(rows_v.at[pl.ds(0, ACCR - HALF)],
                        acc_sh.at[pl.ds(HALF, ACCR - HALF)])

    plsc.subcore_barrier()

    base = cid * HALF

    def _scale(c, buf):
        def scale16(q, c2):
            nvec = norm_v[c2, pl.ds(pl.multiple_of(q * 16, 16), 16)]
            for k in range(16):
                nv = nvec[k]
                r = q * 16 + k
                for j in range(D // 16):
                    buf[r, pl.ds(j * 16, 16)] = buf[r, pl.ds(j * 16, 16)] * nv
            return c2

        lax.fori_loop(0, CH // 16, scale16, c)

    for p in range(NPASS):
        pltpu.sync_copy(src_hbm.at[sid, p], src_v)
        pltpu.sync_copy(dst_hbm.at[sid, p], dst_v)
        pltpu.sync_copy(norm_hbm.at[sid, p], norm_v)

        def localize(c, carry):
            for j in range(CH // 16):
                sl = pl.ds(j * 16, 16)
                loc = dst_v[c, sl] - base
                ok = (loc >= 0) & (loc < HALF)
                dst_v[c, sl] = jnp.where(ok, loc, HALF)
            return carry

        lax.fori_loop(0, NCHUNK, localize, 0)

        pltpu.make_async_copy(g_hbm.at[src_v.at[0]], rows_v, sem).start()
        pltpu.make_async_copy(g_hbm.at[src_v.at[1]], rows2_v, sem2).start()

        def pair(i, carry):
            c0 = 2 * i
            pltpu.make_async_copy(g_hbm.at[src_v.at[c0]], rows_v, sem).wait()
            _scale(c0, rows_v)
            d0 = pltpu.async_copy(rows_v, acc_sh.at[dst_v.at[c0]], ssem,
                                  add=True)
            pltpu.make_async_copy(g_hbm.at[src_v.at[c0 + 1]], rows2_v,
                                  sem2).wait()
            _scale(c0 + 1, rows2_v)
            d1 = pltpu.async_copy(rows2_v, acc_sh.at[dst_v.at[c0 + 1]], ssem2,
                                  add=True)
            d0.wait()
            pltpu.make_async_copy(g_hbm.at[src_v.at[c0 + 2]], rows_v,
                                  sem).start()
            d1.wait()

            @pl.when(c0 + 3 < NCHUNK)
            def _():
                pltpu.make_async_copy(g_hbm.at[src_v.at[c0 + 3]], rows2_v,
                                      sem2).start()

            return carry

        lax.fori_loop(0, (NCHUNK - 1) // 2, pair, 0)
        pltpu.make_async_copy(g_hbm.at[src_v.at[NCHUNK - 1]], rows_v, sem).wait()
        _scale(NCHUNK - 1, rows_v)
        pltpu.sync_copy(rows_v, acc_sh.at[dst_v.at[NCHUNK - 1]], add=True)
    plsc.subcore_barrier()
    pltpu.sync_copy(acc_sh.at[pl.ds(sid * RPS, RPS)],
                    out_hbm.at[cid, pl.ds(sid * RPS, RPS)])


_msg_call = pl.kernel(
    _msg_body,
    out_type=jax.ShapeDtypeStruct((NC, HALF, D), _f32),
    mesh=_MESH,
    scratch_types=[
        pltpu.VMEM((NCHUNK, CH), _i32),
        pltpu.VMEM((NCHUNK, CH), _i32),
        pltpu.VMEM((NCHUNK, CH), _f32),
        pltpu.VMEM((CH, D), _f32),
        pltpu.VMEM((CH, D), _f32),
        pltpu.VMEM_SHARED((ACCR, D), _f32),
        pltpu.SemaphoreType.DMA,
        pltpu.SemaphoreType.DMA,
        pltpu.SemaphoreType.DMA,
        pltpu.SemaphoreType.DMA,
    ],
)


def _tc1_body(x_ref, w_ref, degp_ref, g_ref, dinv_ref):
    g_ref[...] = jnp.dot(x_ref[...], w_ref[...], preferred_element_type=_f32)
    deg = degp_ref[0] + degp_ref[1]
    dinv_ref[...] = jnp.where(deg > 0, lax.rsqrt(deg), 0.0)


def _tc1(x, w, degp):
    return pl.pallas_call(
        _tc1_body,
        out_shape=(jax.ShapeDtypeStruct((NPAD, D), _f32),
                   jax.ShapeDtypeStruct((NPAD // 128, 128), _f32)),
    )(x, w, degp)


def _tc_mid_body(s_ref, b_ref, w_ref, g_ref):
    h = jnp.maximum(s_ref[...] + b_ref[...], 0.0)
    g_ref[...] = jnp.dot(h, w_ref[...], preferred_element_type=_f32)


def _tc_mid(s, b, w):
    return pl.pallas_call(
        _tc_mid_body,
        out_shape=jax.ShapeDtypeStruct((NPAD, D), _f32),
    )(s, b, w)


def _tc_fin_body(s_ref, b_ref, batch_ref, wc_ref, bc_ref, out_ref):
    h = jnp.maximum(s_ref[...] + b_ref[...], 0.0)
    gid = lax.broadcasted_iota(_i32, (G, NPAD), 0)
    m = (gid == batch_ref[...]).astype(_f32)
    sums = jnp.dot(m, h, preferred_element_type=_f32)
    cnt = jnp.maximum(jnp.sum(m, axis=1, keepdims=True), 1.0)
    z = jnp.dot(sums / cnt, wc_ref[...], preferred_element_type=_f32) + bc_ref[...]
    out_ref[...] = jax.nn.sigmoid(z)


def _tc_fin(s, b, batch2, wc, bc):
    return pl.pallas_call(
        _tc_fin_body,
        out_shape=jax.ShapeDtypeStruct((G, 1), _f32),
    )(s, b, batch2, wc, bc)


def kernel(x, edge_index, edge_weight, batch, W1, b1, W2, b2, W3, b3, W4, b4,
           Wc, bc):
    src = edge_index[0]
    dst = edge_index[1]
    xp = jnp.concatenate([x, jnp.zeros((NPAD - N, D), _f32)])
    batchp = jnp.concatenate([batch, jnp.full((NPAD - N,), -1, _i32)])
    loops = jnp.arange(N, dtype=_i32)
    padi = jnp.zeros((EP - E - N,), _i32)
    srcf = jnp.concatenate([src, loops, padi])
    dstf = jnp.concatenate([dst, loops, padi])
    ewf = jnp.concatenate([edge_weight, jnp.ones((N,), _f32),
                           jnp.zeros((EP - E - N,), _f32)])
    srcp = srcf.reshape(NW, NCHUNK, CH)
    dstp = dstf.reshape(NW, NCHUNK, CH)
    ewp = ewf.reshape(NW, NCHUNK, CH)
    src16 = srcf.reshape(NS, NPASS, NCHUNK, CH)
    dst16 = dstf.reshape(NS, NPASS, NCHUNK, CH)

    degp = _deg_call(dstp, ewp)
    g, dinv2 = _tc1(xp, W1, degp.reshape(NC, NPAD // 128, 128))
    dinv = dinv2.reshape(NPAD)
    norm16 = _norm_call(dinv, srcp, dstp, ewp).reshape(NS, NPASS, NCHUNK, CH)

    b1r = b1.reshape(1, D)
    b2r = b2.reshape(1, D)
    b3r = b3.reshape(1, D)
    b4r = b4.reshape(1, D)

    s = _msg_call(g, src16, dst16, norm16).reshape(NPAD, D)
    g = _tc_mid(s, b1r, W2)
    s = _msg_call(g, src16, dst16, norm16).reshape(NPAD, D)
    g = _tc_mid(s, b2r, W3)
    s = _msg_call(g, src16, dst16, norm16).reshape(NPAD, D)
    g = _tc_mid(s, b3r, W4)
    s = _msg_call(g, src16, dst16, norm16).reshape(NPAD, D)
    return _tc_fin(s, b4r, batchp.reshape(1, NPAD), Wc, bc.reshape(1, 1))


# speedup vs baseline: 1.1680x; 1.1680x over previous
import jax
import jax.numpy as jnp
from jax import lax
from jax.experimental import pallas as pl
from jax.experimental.pallas import tpu as pltpu
from jax.experimental.pallas import tpu_sc as plsc

N = 10000
E = 320000
D = 128
G = 64
NC = 2
NS = 16
NW = NC * NS
CH = 128
NCHUNK = 81
EPW = CH * NCHUNK
EP = NW * EPW
NPASS = 2
NPAD = 10240
HALF = NPAD // 2
ACCR = HALF + 8
RPC = NPAD // NS
RPS = HALF // NS

_f32 = jnp.float32
_i32 = jnp.int32

_MESH = plsc.VectorSubcoreMesh(core_axis_name="c", subcore_axis_name="s")


def _wid():
    return lax.axis_index("c") * NS + lax.axis_index("s")


def _deg_body(dst_hbm, ew_hbm, out_hbm, dst_v, ew_v, zrow_v, acc_sh):
    cid = lax.axis_index("c")
    sid = lax.axis_index("s")
    wid = _wid()

    def zrow(i, c):
        zrow_v[pl.ds(pl.multiple_of(i * 16, 16), 16)] = jnp.zeros((16,), _f32)
        return c

    lax.fori_loop(0, RPC // 16, zrow, 0)
    pltpu.sync_copy(zrow_v, acc_sh.at[pl.ds(sid * RPC, RPC)])
    plsc.subcore_barrier()

    pltpu.sync_copy(dst_hbm.at[wid], dst_v)
    pltpu.sync_copy(ew_hbm.at[wid], ew_v)

    def chunk(c, carry):
        pltpu.sync_copy(ew_v.at[c], acc_sh.at[dst_v.at[c]], add=True)
        return carry

    lax.fori_loop(0, NCHUNK, chunk, 0)
    plsc.subcore_barrier()
    pltpu.sync_copy(acc_sh.at[pl.ds(sid * RPC, RPC)],
                    out_hbm.at[cid, pl.ds(sid * RPC, RPC)])


_deg_call = pl.kernel(
    _deg_body,
    out_type=jax.ShapeDtypeStruct((NC, NPAD), _f32),
    mesh=_MESH,
    scratch_types=[
        pltpu.VMEM((NCHUNK, CH), _i32),
        pltpu.VMEM((NCHUNK, CH), _f32),
        pltpu.VMEM((RPC,), _f32),
        pltpu.VMEM_SHARED((NPAD,), _f32),
    ],
)


def _norm_body(dinv_hbm, src_hbm, dst_hbm, ew_hbm, norm_hbm,
               src_v, dst_v, ew_v, norm_v, ds_v, dd_v, sem, sem2):
    wid = _wid()
    pltpu.sync_copy(src_hbm.at[wid], src_v)
    pltpu.sync_copy(dst_hbm.at[wid], dst_v)
    pltpu.sync_copy(ew_hbm.at[wid], ew_v)

    def chunk(c, carry):
        a = pltpu.async_copy(dinv_hbm.at[src_v.at[c]], ds_v, sem)
        b = pltpu.async_copy(dinv_hbm.at[dst_v.at[c]], dd_v, sem2)
        a.wait()
        b.wait()
        for j in range(CH // 16):
            sl = pl.ds(j * 16, 16)
            norm_v[c, sl] = ds_v[sl] * ew_v[c, sl] * dd_v[sl]
        return carry

    lax.fori_loop(0, NCHUNK, chunk, 0)
    pltpu.sync_copy(norm_v, norm_hbm.at[wid])


_norm_call = pl.kernel(
    _norm_body,
    out_type=jax.ShapeDtypeStruct((NW, NCHUNK, CH), _f32),
    mesh=_MESH,
    scratch_types=[
        pltpu.VMEM((NCHUNK, CH), _i32),
        pltpu.VMEM((NCHUNK, CH), _i32),
        pltpu.VMEM((NCHUNK, CH), _f32),
        pltpu.VMEM((NCHUNK, CH), _f32),
        pltpu.VMEM((CH,), _f32),
        pltpu.VMEM((CH,), _f32),
        pltpu.SemaphoreType.DMA,
        pltpu.SemaphoreType.DMA,
    ],
)


WREG = NCHUNK * CH
SREG = WREG * 2 + 128


def _part_body(src_hbm, dst_hbm, norm_hbm, psrc_hbm, pdst_hbm, pnorm_hbm,
               cnt_hbm, src_v, dst_v, norm_v, slot_v, cnt_v,
               stg_src, stg_dst, stg_norm):
    cid = lax.axis_index("c")
    sid = lax.axis_index("s")
    pltpu.sync_copy(src_hbm.at[sid, cid], src_v)
    pltpu.sync_copy(dst_hbm.at[sid, cid], dst_v)
    pltpu.sync_copy(norm_hbm.at[sid, cid], norm_v)

    sbase = sid * SREG
    lane16 = lax.broadcasted_iota(_i32, (16,), 0)

    def chunk(c, off):
        offa, offb = off
        for j in range(CH // 16):
            sl = pl.ds(j * 16, 16)
            d = dst_v[c, sl]
            m = d < HALF
            mi = jnp.where(m, 1, 0)
            csa = jnp.zeros((16,), _i32)
            for k in range(16):
                csa = csa + jnp.where(lane16 >= k, mi[k], 0)
            csb = (lane16 + 1) - csa
            pca = csa[15]
            slot = jnp.where(m, sbase + offa + csa - 1,
                             sbase + WREG + offb + csb - 1)
            slot_v[c, sl] = slot
            dst_v[c, sl] = jnp.where(m, d, d - HALF)
            offa = offa + pca
            offb = offb + (16 - pca)
        pltpu.sync_copy(src_v.at[c], stg_src.at[slot_v.at[c]])
        pltpu.sync_copy(dst_v.at[c], stg_dst.at[slot_v.at[c]])
        pltpu.sync_copy(norm_v.at[c], stg_norm.at[slot_v.at[c]])
        return (offa, offb)

    zoff = jnp.zeros((16,), _i32)
    offa, offb = lax.fori_loop(0, NCHUNK, chunk, (zoff, zoff))

    trash = sbase + 2 * WREG + lane16
    counts = []
    for side, off in ((0, offa), (1, offb)):
        pad = lax.rem(CH - lax.rem(off, CH), jnp.full((16,), CH, _i32))
        counts.append(lax.div(off + pad, jnp.full((16,), CH, _i32)))
        row = side
        for j in range(CH // 16):
            sl = pl.ds(j * 16, 16)
            lane = lane16 + (j * 16)
            sel = lane < pad
            slot_v[row, sl] = jnp.where(
                sel, sbase + side * WREG + off + lane, trash)
            src_v[row, sl] = jnp.zeros((16,), _i32)
            dst_v[row, sl] = jnp.full((16,), HALF, _i32)
            norm_v[row, sl] = jnp.zeros((16,), _f32)
        pltpu.sync_copy(src_v.at[row], stg_src.at[slot_v.at[row]])
        pltpu.sync_copy(dst_v.at[row], stg_dst.at[slot_v.at[row]])
        pltpu.sync_copy(norm_v.at[row], stg_norm.at[slot_v.at[row]])

    for side in (0, 1):
        hoff = (((side * NS + sid) * NPASS) + cid) * WREG
        soff = sbase + side * WREG
        pltpu.sync_copy(stg_src.at[pl.ds(soff, WREG)],
                        psrc_hbm.at[pl.ds(hoff, WREG)])
        pltpu.sync_copy(stg_dst.at[pl.ds(soff, WREG)],
                        pdst_hbm.at[pl.ds(hoff, WREG)])
        pltpu.sync_copy(stg_norm.at[pl.ds(soff, WREG)],
                        pnorm_hbm.at[pl.ds(hoff, WREG)])
        for j in range(CH // 16):
            cnt_v[pl.ds(j * 16, 16)] = counts[side]
        pltpu.sync_copy(cnt_v, cnt_hbm.at[side, sid, cid])


_PFLAT = NC * NS * NPASS * WREG

_part_call = pl.kernel(
    _part_body,
    out_type=(jax.ShapeDtypeStruct((_PFLAT,), _i32),
              jax.ShapeDtypeStruct((_PFLAT,), _i32),
              jax.ShapeDtypeStruct((_PFLAT,), _f32),
              jax.ShapeDtypeStruct((NC, NS, NPASS, CH), _i32)),
    mesh=_MESH,
    scratch_types=[
        pltpu.VMEM((NCHUNK, CH), _i32),
        pltpu.VMEM((NCHUNK, CH), _i32),
        pltpu.VMEM((NCHUNK, CH), _f32),
        pltpu.VMEM((NCHUNK, CH), _i32),
        pltpu.VMEM((CH,), _i32),
        pltpu.VMEM_SHARED((NS * SREG,), _i32),
        pltpu.VMEM_SHARED((NS * SREG,), _i32),
        pltpu.VMEM_SHARED((NS * SREG,), _f32),
    ],
)

def _msg_body(g_hbm, src_hbm, dst_hbm, norm_hbm, cnt_hbm, out_hbm,
              src_v, dst_v, norm_v, rows_v, rows2_v, cnt_v, acc_sh, sem, sem2):
    cid = lax.axis_index("c")
    sid = lax.axis_index("s")

    def zrow(r, c):
        for j in range(D // 16):
            rows_v[r, pl.ds(j * 16, 16)] = jnp.zeros((16,), _f32)
        return c

    lax.fori_loop(0, CH, zrow, 0)
    for k in range(2):
        pltpu.sync_copy(rows_v, acc_sh.at[pl.ds(sid * RPS + k * CH, CH)])
    pltpu.sync_copy(rows_v.at[pl.ds(0, RPS - 2 * CH)],
                    acc_sh.at[pl.ds(sid * RPS + 2 * CH, RPS - 2 * CH)])

    @pl.when(sid == 0)
    def _():
        pltpu.sync_copy(rows_v.at[pl.ds(0, ACCR - HALF)],
                        acc_sh.at[pl.ds(HALF, ACCR - HALF)])

    plsc.subcore_barrier()

    def _scale(c, buf):
        def scale16(q, c2):
            nvec = norm_v[c2, pl.ds(pl.multiple_of(q * 16, 16), 16)]
            for k in range(16):
                nv = nvec[k]
                r = q * 16 + k
                for j in range(D // 16):
                    buf[r, pl.ds(j * 16, 16)] = buf[r, pl.ds(j * 16, 16)] * nv
            return c2

        lax.fori_loop(0, CH // 16, scale16, c)

    for p in range(NPASS):
        pltpu.sync_copy(src_hbm.at[cid, sid, p], src_v)
        pltpu.sync_copy(dst_hbm.at[cid, sid, p], dst_v)
        pltpu.sync_copy(norm_hbm.at[cid, sid, p], norm_v)
        pltpu.sync_copy(cnt_hbm.at[cid, sid, p], cnt_v)
        cnt = cnt_v[pl.ds(0, 16)][0]

        @pl.when(cnt > 0)
        def _():
            pltpu.make_async_copy(g_hbm.at[src_v.at[0]], rows_v, sem).start()

        def pair(i, carry):
            c0 = 2 * i

            @pl.when(c0 + 1 < cnt)
            def _():
                pltpu.make_async_copy(g_hbm.at[src_v.at[c0 + 1]], rows2_v,
                                      sem2).start()

            @pl.when(c0 < cnt)
            def _():
                pltpu.make_async_copy(g_hbm.at[src_v.at[c0]], rows_v,
                                      sem).wait()
                _scale(c0, rows_v)
                pltpu.sync_copy(rows_v, acc_sh.at[dst_v.at[c0]], add=True)

            @pl.when(c0 + 2 < cnt)
            def _():
                pltpu.make_async_copy(g_hbm.at[src_v.at[c0 + 2]], rows_v,
                                      sem).start()

            @pl.when(c0 + 1 < cnt)
            def _():
                pltpu.make_async_copy(g_hbm.at[src_v.at[c0 + 1]], rows2_v,
                                      sem2).wait()
                _scale(c0 + 1, rows2_v)
                pltpu.sync_copy(rows2_v, acc_sh.at[dst_v.at[c0 + 1]], add=True)

            return carry

        lax.fori_loop(0, (NCHUNK + 1) // 2, pair, 0)
    plsc.subcore_barrier()
    pltpu.sync_copy(acc_sh.at[pl.ds(sid * RPS, RPS)],
                    out_hbm.at[cid, pl.ds(sid * RPS, RPS)])


_msg_call = pl.kernel(
    _msg_body,
    out_type=jax.ShapeDtypeStruct((NC, HALF, D), _f32),
    mesh=_MESH,
    scratch_types=[
        pltpu.VMEM((NCHUNK, CH), _i32),
        pltpu.VMEM((NCHUNK, CH), _i32),
        pltpu.VMEM((NCHUNK, CH), _f32),
        pltpu.VMEM((CH, D), _f32),
        pltpu.VMEM((CH, D), _f32),
        pltpu.VMEM((CH,), _i32),
        pltpu.VMEM_SHARED((ACCR, D), _f32),
        pltpu.SemaphoreType.DMA,
        pltpu.SemaphoreType.DMA,
    ],
)


def _tc1_body(x_ref, w_ref, degp_ref, g_ref, dinv_ref):
    g_ref[...] = jnp.dot(x_ref[...], w_ref[...], preferred_element_type=_f32)
    deg = degp_ref[0] + degp_ref[1]
    dinv_ref[...] = jnp.where(deg > 0, lax.rsqrt(deg), 0.0)


def _tc1(x, w, degp):
    return pl.pallas_call(
        _tc1_body,
        out_shape=(jax.ShapeDtypeStruct((NPAD, D), _f32),
                   jax.ShapeDtypeStruct((NPAD // 128, 128), _f32)),
    )(x, w, degp)


def _tc_mid_body(s_ref, b_ref, w_ref, g_ref):
    h = jnp.maximum(s_ref[...] + b_ref[...], 0.0)
    g_ref[...] = jnp.dot(h, w_ref[...], preferred_element_type=_f32)


def _tc_mid(s, b, w):
    return pl.pallas_call(
        _tc_mid_body,
        out_shape=jax.ShapeDtypeStruct((NPAD, D), _f32),
    )(s, b, w)


def _tc_fin_body(s_ref, b_ref, batch_ref, wc_ref, bc_ref, out_ref):
    h = jnp.maximum(s_ref[...] + b_ref[...], 0.0)
    gid = lax.broadcasted_iota(_i32, (G, NPAD), 0)
    m = (gid == batch_ref[...]).astype(_f32)
    sums = jnp.dot(m, h, preferred_element_type=_f32)
    cnt = jnp.maximum(jnp.sum(m, axis=1, keepdims=True), 1.0)
    z = jnp.dot(sums / cnt, wc_ref[...], preferred_element_type=_f32) + bc_ref[...]
    out_ref[...] = jax.nn.sigmoid(z)


def _tc_fin(s, b, batch2, wc, bc):
    return pl.pallas_call(
        _tc_fin_body,
        out_shape=jax.ShapeDtypeStruct((G, 1), _f32),
    )(s, b, batch2, wc, bc)


def kernel(x, edge_index, edge_weight, batch, W1, b1, W2, b2, W3, b3, W4, b4,
           Wc, bc):
    src = edge_index[0]
    dst = edge_index[1]
    xp = jnp.concatenate([x, jnp.zeros((NPAD - N, D), _f32)])
    batchp = jnp.concatenate([batch, jnp.full((NPAD - N,), -1, _i32)])
    loops = jnp.arange(N, dtype=_i32)
    padi = jnp.zeros((EP - E - N,), _i32)
    srcf = jnp.concatenate([src, loops, padi])
    dstf = jnp.concatenate([dst, loops, padi])
    ewf = jnp.concatenate([edge_weight, jnp.ones((N,), _f32),
                           jnp.zeros((EP - E - N,), _f32)])
    srcp = srcf.reshape(NW, NCHUNK, CH)
    dstp = dstf.reshape(NW, NCHUNK, CH)
    ewp = ewf.reshape(NW, NCHUNK, CH)
    src16 = srcf.reshape(NS, NPASS, NCHUNK, CH)
    dst16 = dstf.reshape(NS, NPASS, NCHUNK, CH)
    norm_shape = (NS, NPASS, NCHUNK, CH)

    degp = _deg_call(dstp, ewp)
    g, dinv2 = _tc1(xp, W1, degp.reshape(NC, NPAD // 128, 128))
    dinv = dinv2.reshape(NPAD)
    norm16 = _norm_call(dinv, srcp, dstp, ewp).reshape(norm_shape)
    psrc, pdst, pnorm, pcnt = _part_call(src16, dst16, norm16)
    pshape = (NC, NS, NPASS, NCHUNK, CH)
    psrc = psrc.reshape(pshape)
    pdst = pdst.reshape(pshape)
    pnorm = pnorm.reshape(pshape)

    b1r = b1.reshape(1, D)
    b2r = b2.reshape(1, D)
    b3r = b3.reshape(1, D)
    b4r = b4.reshape(1, D)

    s = _msg_call(g, psrc, pdst, pnorm, pcnt).reshape(NPAD, D)
    g = _tc_mid(s, b1r, W2)
    s = _msg_call(g, psrc, pdst, pnorm, pcnt).reshape(NPAD, D)
    g = _tc_mid(s, b2r, W3)
    s = _msg_call(g, psrc, pdst, pnorm, pcnt).reshape(NPAD, D)
    g = _tc_mid(s, b3r, W4)
    s = _msg_call(g, psrc, pdst, pnorm, pcnt).reshape(NPAD, D)
    return _tc_fin(s, b4r, batchp.reshape(1, NPAD), Wc, bc.reshape(1, 1))
